# baseline (speedup 1.0000x reference)
import functools

import jax
import jax.numpy as jnp
from jax.experimental import pallas as pl
from jax.experimental.pallas import tpu as pltpu

_EPS = 1e-5
_CH = 1024

_OMICS_ROW = {"mrna": 0, "mirna": 1, "methylation": 2}


def _pass1_kernel(a_ref, dis_ref, slab_ref, acc_ref):
    i = pl.program_id(0)
    a = a_ref[...]
    slab_ref[...] = a[:, 0:128]
    col = jnp.sum(a, axis=0, keepdims=True)

    @pl.when(i == 0)
    def _init():
        acc_ref[...] = jnp.zeros_like(acc_ref)

    acc_ref[...] += col

    @pl.when(i == pl.num_programs(0) - 1)
    def _fin():
        dis_ref[...] = jax.lax.rsqrt(acc_ref[...] + 1.0)


def _prep_kernel(x_ref, w1_ref, dis_ref, v_ref):
    h = jnp.dot(x_ref[...], w1_ref[...], preferred_element_type=jnp.float32,
                precision=jax.lax.Precision.HIGHEST)
    v_ref[...] = dis_ref[...] * h


def _pass2_kernel(a_ref, v_ref, aux_ref, w2_ref, b1_ref, b2_ref,
                  xg_ref, r_ref, *, rb, fin_chunk):
    i = pl.program_id(0)

    @pl.when(i == 0)
    def _init():
        r_ref[...] = jnp.zeros_like(r_ref)

    v_blk = v_ref[pl.ds(i * rb, rb), :]
    r_ref[...] += jax.lax.dot_general(
        a_ref[...], v_blk, (((0,), (0,)), ((), ())),
        preferred_element_type=jnp.float32, precision=jax.lax.Precision.HIGHEST)

    @pl.when(i == pl.num_programs(0) - 1)
    def _fin():
        n = r_ref.shape[0]
        lat = r_ref.shape[1]
        agg8 = jnp.zeros((8, lat), jnp.float32)
        u28 = jnp.zeros((8, lat), jnp.float32)
        w2 = w2_ref[...]
        b1 = b1_ref[...]
        for c in range(0, n, fin_chunk):
            sl = pl.ds(c, fin_chunk)
            aux = aux_ref[sl, :]
            dis_c = aux[:, 8:9]
            slab_c = aux[:, 0:8]
            x1 = jnp.maximum(dis_c * (r_ref[sl, :] + v_ref[sl, :]) + b1, 0.0)
            h2 = jnp.dot(x1, w2, preferred_element_type=jnp.float32,
                         precision=jax.lax.Precision.HIGHEST)
            u2 = dis_c * h2
            agg8 = agg8 + jax.lax.dot_general(
                slab_c, u2, (((0,), (0,)), ((), ())),
                preferred_element_type=jnp.float32,
                precision=jax.lax.Precision.HIGHEST)
            if c == 0:
                u28 = u2[0:8, :]
        r8 = jax.lax.broadcasted_iota(jnp.int32, (8, 8), 0)
        c8 = jax.lax.broadcasted_iota(jnp.int32, (8, 8), 1)
        d8 = jnp.where(r8 == c8, 1.0, 0.0) * aux_ref[0:8, 8:9]
        pre = jnp.dot(d8, agg8 + u28, preferred_element_type=jnp.float32,
                      precision=jax.lax.Precision.HIGHEST)
        xg_ref[...] = jnp.maximum(pre + b2_ref[...], 0.0)


def _gen_kernel(xg_ref, w1_ref, b1_ref, g1_ref, be1_ref, rm1_ref, rv1_ref,
                w2_ref, b2_ref, g2_ref, be2_ref, rm2_ref, rv2_ref, o_ref):
    vrow = xg_ref[...]
    h = jnp.dot(vrow, w1_ref[...], preferred_element_type=jnp.float32,
                precision=jax.lax.Precision.HIGHEST)
    h = h + b1_ref[...]
    h = (h - rm1_ref[...]) * jax.lax.rsqrt(rv1_ref[...] + _EPS)
    h = h * g1_ref[...] + be1_ref[...]
    h = jnp.maximum(h, 0.0)
    o = jnp.dot(h, w2_ref[...], preferred_element_type=jnp.float32,
                precision=jax.lax.Precision.HIGHEST)
    o = o + b2_ref[...]
    o = (o - rm2_ref[...]) * jax.lax.rsqrt(rv2_ref[...] + _EPS)
    o_ref[...] = o * g2_ref[...] + be2_ref[...]


def _full(shape):
    return pl.BlockSpec(shape, lambda *_: tuple(0 for _ in shape))


def kernel(latent_vectors, adjacency_matrix, gcn_params, gen_params):
    n, lat = latent_vectors.shape
    rb = 80
    assert n % rb == 0
    g = n // rb

    dis_row, slab = pl.pallas_call(
        _pass1_kernel,
        grid=(g,),
        in_specs=[pl.BlockSpec((rb, n), lambda i: (i, 0))],
        out_specs=[pl.BlockSpec((1, n), lambda i: (0, 0)),
                   pl.BlockSpec((rb, 128), lambda i: (i, 0))],
        out_shape=[jax.ShapeDtypeStruct((1, n), jnp.float32),
                   jax.ShapeDtypeStruct((n, 128), jnp.float32)],
        scratch_shapes=[pltpu.VMEM((1, n), jnp.float32)],
    )(adjacency_matrix)

    dis_col = dis_row.reshape(n, 1)
    (w1, b1), (w2, b2) = gcn_params
    b1r = b1.reshape(1, lat)
    b2r = b2.reshape(1, lat)
    aux = jnp.concatenate([slab[:, 0:8], dis_col], axis=1)

    v = pl.pallas_call(
        _prep_kernel,
        in_specs=[_full((n, lat)), _full((lat, lat)), _full((n, 1))],
        out_specs=_full((n, lat)),
        out_shape=jax.ShapeDtypeStruct((n, lat), jnp.float32),
    )(latent_vectors, w1, dis_col)

    xg8 = pl.pallas_call(
        functools.partial(_pass2_kernel, rb=rb, fin_chunk=500),
        grid=(g,),
        in_specs=[pl.BlockSpec((rb, n), lambda i: (i, 0)),
                  pl.BlockSpec((n, lat), lambda i: (0, 0)),
                  pl.BlockSpec((n, 9), lambda i: (0, 0)),
                  pl.BlockSpec((lat, lat), lambda i: (0, 0)),
                  pl.BlockSpec((1, lat), lambda i: (0, 0)),
                  pl.BlockSpec((1, lat), lambda i: (0, 0))],
        out_specs=pl.BlockSpec((8, lat), lambda i: (0, 0)),
        out_shape=jax.ShapeDtypeStruct((8, lat), jnp.float32),
        scratch_shapes=[pltpu.VMEM((n, lat), jnp.float32)],
    )(adjacency_matrix, v, aux, w2, b1r, b2r)

    out = {}
    for name, p in gen_params.items():
        hid = p["W1"].shape[1]
        dim = p["W2"].shape[1]
        dpad = ((dim + _CH - 1) // _CH) * _CH
        vecs1 = [p[k].reshape(1, hid) for k in ("b1", "g1", "be1", "rm1", "rv1")]
        w2p = jnp.pad(p["W2"], ((0, 0), (0, dpad - dim)))
        vecs2p = [jnp.pad(p[k].reshape(1, dim), ((0, 0), (0, dpad - dim)),
                          constant_values=(1.0 if k == "rv2" else 0.0))
                  for k in ("b2", "g2", "be2", "rm2", "rv2")]
        o = pl.pallas_call(
            _gen_kernel,
            grid=(dpad // _CH,),
            in_specs=[pl.BlockSpec((8, lat), lambda j: (0, 0)),
                      pl.BlockSpec((lat, hid), lambda j: (0, 0))]
                     + [pl.BlockSpec((1, hid), lambda j: (0, 0))] * 5
                     + [pl.BlockSpec((hid, _CH), lambda j: (0, j))]
                     + [pl.BlockSpec((1, _CH), lambda j: (0, j))] * 5,
            out_specs=pl.BlockSpec((8, _CH), lambda j: (0, j)),
            out_shape=jax.ShapeDtypeStruct((8, dpad), jnp.float32),
        )(xg8, p["W1"], *vecs1, w2p, *vecs2p)
        row = _OMICS_ROW.get(name, 0)
        out[name] = jax.lax.slice(o, (row, 0), (row + 1, dim)).reshape(dim)
    return out


# speedup vs baseline: 2.0356x; 2.0356x over previous
import functools

import jax
import jax.numpy as jnp
from jax.experimental import pallas as pl
from jax.experimental.pallas import tpu as pltpu

_EPS = 1e-5
_CH = 1024

_OMICS_ROW = {"mrna": 0, "mirna": 1, "methylation": 2}


def _pass1_kernel(a_ref, dis_ref, slab_ref, abf_ref, acc_ref):
    i = pl.program_id(0)
    a = a_ref[...]
    slab_ref[...] = a[:, 0:128]
    abf_ref[...] = a.astype(jnp.bfloat16)
    col = jnp.sum(a, axis=0, keepdims=True)

    @pl.when(i == 0)
    def _init():
        acc_ref[...] = jnp.zeros_like(acc_ref)

    acc_ref[...] += col

    @pl.when(i == pl.num_programs(0) - 1)
    def _fin():
        dis_ref[...] = jax.lax.rsqrt(acc_ref[...] + 1.0)


def _prep_kernel(x_ref, w1_ref, dis_ref, v_ref):
    h = jnp.dot(x_ref[...], w1_ref[...], preferred_element_type=jnp.float32,
                precision=jax.lax.Precision.HIGHEST)
    v_ref[...] = dis_ref[...] * h


def _pass2_kernel(a_ref, v_ref, aux_ref, w2_ref, b1_ref, b2_ref,
                  xg_ref, r_ref, *, rb, fin_chunk):
    i = pl.program_id(0)

    @pl.when(i == 0)
    def _init():
        r_ref[...] = jnp.zeros_like(r_ref)

    v_blk = v_ref[pl.ds(i * rb, rb), :].astype(jnp.bfloat16)
    r_ref[...] += jax.lax.dot_general(
        a_ref[...], v_blk, (((0,), (0,)), ((), ())),
        preferred_element_type=jnp.float32)

    @pl.when(i == pl.num_programs(0) - 1)
    def _fin():
        n = r_ref.shape[0]
        lat = r_ref.shape[1]
        agg8 = jnp.zeros((8, lat), jnp.float32)
        u28 = jnp.zeros((8, lat), jnp.float32)
        w2 = w2_ref[...]
        b1 = b1_ref[...]
        for c in range(0, n, fin_chunk):
            sl = pl.ds(c, fin_chunk)
            aux = aux_ref[sl, :]
            dis_c = aux[:, 8:9]
            slab_c = aux[:, 0:8]
            x1 = jnp.maximum(dis_c * (r_ref[sl, :] + v_ref[sl, :]) + b1, 0.0)
            h2 = jnp.dot(x1, w2, preferred_element_type=jnp.float32,
                         precision=jax.lax.Precision.HIGHEST)
            u2 = dis_c * h2
            agg8 = agg8 + jax.lax.dot_general(
                slab_c, u2, (((0,), (0,)), ((), ())),
                preferred_element_type=jnp.float32,
                precision=jax.lax.Precision.HIGHEST)
            if c == 0:
                u28 = u2[0:8, :]
        r8 = jax.lax.broadcasted_iota(jnp.int32, (8, 8), 0)
        c8 = jax.lax.broadcasted_iota(jnp.int32, (8, 8), 1)
        d8 = jnp.where(r8 == c8, 1.0, 0.0) * aux_ref[0:8, 8:9]
        pre = jnp.dot(d8, agg8 + u28, preferred_element_type=jnp.float32,
                      precision=jax.lax.Precision.HIGHEST)
        xg_ref[...] = jnp.maximum(pre + b2_ref[...], 0.0)


def _gen_kernel(xg_ref, w1_ref, b1_ref, g1_ref, be1_ref, rm1_ref, rv1_ref,
                w2_ref, b2_ref, g2_ref, be2_ref, rm2_ref, rv2_ref, o_ref):
    vrow = xg_ref[...]
    h = jnp.dot(vrow, w1_ref[...], preferred_element_type=jnp.float32,
                precision=jax.lax.Precision.HIGHEST)
    h = h + b1_ref[...]
    h = (h - rm1_ref[...]) * jax.lax.rsqrt(rv1_ref[...] + _EPS)
    h = h * g1_ref[...] + be1_ref[...]
    h = jnp.maximum(h, 0.0)
    o = jnp.dot(h, w2_ref[...], preferred_element_type=jnp.float32,
                precision=jax.lax.Precision.HIGHEST)
    o = o + b2_ref[...]
    o = (o - rm2_ref[...]) * jax.lax.rsqrt(rv2_ref[...] + _EPS)
    o_ref[...] = o * g2_ref[...] + be2_ref[...]


def _full(shape):
    return pl.BlockSpec(shape, lambda *_: tuple(0 for _ in shape))


def kernel(latent_vectors, adjacency_matrix, gcn_params, gen_params):
    n, lat = latent_vectors.shape
    rb1 = 200
    rb = 400
    assert n % rb1 == 0 and n % rb == 0
    g1 = n // rb1
    g = n // rb

    dis_row, slab, abf = pl.pallas_call(
        _pass1_kernel,
        grid=(g1,),
        in_specs=[pl.BlockSpec((rb1, n), lambda i: (i, 0))],
        out_specs=[pl.BlockSpec((1, n), lambda i: (0, 0)),
                   pl.BlockSpec((rb1, 128), lambda i: (i, 0)),
                   pl.BlockSpec((rb1, n), lambda i: (i, 0))],
        out_shape=[jax.ShapeDtypeStruct((1, n), jnp.float32),
                   jax.ShapeDtypeStruct((n, 128), jnp.float32),
                   jax.ShapeDtypeStruct((n, n), jnp.bfloat16)],
        scratch_shapes=[pltpu.VMEM((1, n), jnp.float32)],
    )(adjacency_matrix)

    dis_col = dis_row.reshape(n, 1)
    (w1, b1), (w2, b2) = gcn_params
    b1r = b1.reshape(1, lat)
    b2r = b2.reshape(1, lat)
    aux = jnp.concatenate([slab[:, 0:8], dis_col], axis=1)

    v = pl.pallas_call(
        _prep_kernel,
        in_specs=[_full((n, lat)), _full((lat, lat)), _full((n, 1))],
        out_specs=_full((n, lat)),
        out_shape=jax.ShapeDtypeStruct((n, lat), jnp.float32),
    )(latent_vectors, w1, dis_col)

    xg8 = pl.pallas_call(
        functools.partial(_pass2_kernel, rb=rb, fin_chunk=500),
        grid=(g,),
        in_specs=[pl.BlockSpec((rb, n), lambda i: (i, 0)),
                  pl.BlockSpec((n, lat), lambda i: (0, 0)),
                  pl.BlockSpec((n, 9), lambda i: (0, 0)),
                  pl.BlockSpec((lat, lat), lambda i: (0, 0)),
                  pl.BlockSpec((1, lat), lambda i: (0, 0)),
                  pl.BlockSpec((1, lat), lambda i: (0, 0))],
        out_specs=pl.BlockSpec((8, lat), lambda i: (0, 0)),
        out_shape=jax.ShapeDtypeStruct((8, lat), jnp.float32),
        scratch_shapes=[pltpu.VMEM((n, lat), jnp.float32)],
    )(abf, v, aux, w2, b1r, b2r)

    out = {}
    for name, p in gen_params.items():
        hid = p["W1"].shape[1]
        dim = p["W2"].shape[1]
        dpad = ((dim + _CH - 1) // _CH) * _CH
        vecs1 = [p[k].reshape(1, hid) for k in ("b1", "g1", "be1", "rm1", "rv1")]
        w2p = jnp.pad(p["W2"], ((0, 0), (0, dpad - dim)))
        vecs2p = [jnp.pad(p[k].reshape(1, dim), ((0, 0), (0, dpad - dim)),
                          constant_values=(1.0 if k == "rv2" else 0.0))
                  for k in ("b2", "g2", "be2", "rm2", "rv2")]
        o = pl.pallas_call(
            _gen_kernel,
            grid=(dpad // _CH,),
            in_specs=[pl.BlockSpec((8, lat), lambda j: (0, 0)),
                      pl.BlockSpec((lat, hid), lambda j: (0, 0))]
                     + [pl.BlockSpec((1, hid), lambda j: (0, 0))] * 5
                     + [pl.BlockSpec((hid, _CH), lambda j: (0, j))]
                     + [pl.BlockSpec((1, _CH), lambda j: (0, j))] * 5,
            out_specs=pl.BlockSpec((8, _CH), lambda j: (0, j)),
            out_shape=jax.ShapeDtypeStruct((8, dpad), jnp.float32),
        )(xg8, p["W1"], *vecs1, w2p, *vecs2p)
        row = _OMICS_ROW.get(name, 0)
        out[name] = jax.lax.slice(o, (row, 0), (row + 1, dim)).reshape(dim)
    return out
